# baseline (speedup 1.0000x reference)
import functools

import jax
import jax.numpy as jnp
from jax import lax
from jax.experimental import pallas as pl
from jax.experimental.pallas import tpu as pltpu
from jax.experimental.pallas import tpu_sc as plsc

NC, NS, L = 2, 16, 16
NW = NC * NS
ROWS, COLS = 2048, 4096
R_TC = 1792
R_SC = ROWS - R_TC
ROWS_W = R_SC // NW
BR, BC = 8, 1024
NBLK = (ROWS_W // BR) * (COLS // BC)
NSLOT = 4
TBR = 448

_mesh = plsc.VectorSubcoreMesh(
    core_axis_name="c", subcore_axis_name="s", num_cores=NC, num_subcores=NS
)


@functools.partial(
    pl.kernel,
    mesh=_mesh,
    out_type=jax.ShapeDtypeStruct((R_SC, COLS), jnp.float32),
    scratch_types=[
        pltpu.VMEM((NSLOT, BR, BC), jnp.float32),
        pltpu.VMEM((NSLOT, BR, BC), jnp.int32),
        pltpu.VMEM((NSLOT, BR, BC), jnp.float32),
    ]
    + [pltpu.SemaphoreType.DMA] * (3 * NSLOT),
    compiler_params=pltpu.CompilerParams(use_tc_tiling_on_sc=True),
)
def _cordiv_sc(div_hbm, dsr_hbm, out_hbm, a_v, b_v, o_v, *sems):
    wid = lax.axis_index("s") * NC + lax.axis_index("c")
    row_base = wid * ROWS_W
    sa = sems[0:NSLOT]
    sb = sems[NSLOT : 2 * NSLOT]
    so = sems[2 * NSLOT : 3 * NSLOT]
    zeros = jnp.zeros((L,), jnp.float32)

    def blk(g):
        r0 = row_base + (g // (COLS // BC)) * BR
        c0 = (g % (COLS // BC)) * BC
        return r0, pl.ds(c0, BC)

    def in_copies(g, s):
        r0, c = blk(g)
        rin = pl.ds(R_TC + r0, BR)
        ca = pltpu.async_copy(div_hbm.at[rin, c], a_v.at[s], sa[s])
        cb = pltpu.async_copy(dsr_hbm.at[rin, c], b_v.at[s], sb[s])
        return ca, cb

    def out_copy(g, s):
        r0, c = blk(g)
        return pltpu.async_copy(o_v.at[s], out_hbm.at[pl.ds(r0, BR), c], so[s])

    pend_in = [None] * NSLOT
    pend_out = [None] * NSLOT
    for g in range(min(NSLOT, NBLK)):
        pend_in[g] = in_copies(g, g)
    for g in range(NBLK):
        s = g % NSLOT
        ca, cb = pend_in[s]
        ca.wait()
        cb.wait()
        if pend_out[s] is not None:
            pend_out[s].wait()
        a_s = a_v.at[s]
        b_s = b_v.at[s]
        o_s = o_v.at[s]

        @plsc.parallel_loop(0, BC, step=L, unroll=2)
        def _(i):
            for r in range(BR):
                d = a_s[r, pl.ds(i, L)]
                q = b_s[r, pl.ds(i, L)]
                o_s[r, pl.ds(i, L)] = jnp.where(q == 1, d, zeros)

        pend_out[s] = out_copy(g, s)
        if g + NSLOT < NBLK:
            pend_in[s] = in_copies(g + NSLOT, s)

    for s in range(NSLOT):
        if pend_out[s] is not None:
            pend_out[s].wait()


def _tc_body(d_ref, q_ref, o_ref):
    o_ref[...] = jnp.where(q_ref[...] == 1, d_ref[...], 0.0)


_cordiv_tc = pl.pallas_call(
    _tc_body,
    grid=(R_TC // TBR,),
    in_specs=[
        pl.BlockSpec((TBR, COLS), lambda i: (i, 0)),
        pl.BlockSpec((TBR, COLS), lambda i: (i, 0)),
    ],
    out_specs=pl.BlockSpec((TBR, COLS), lambda i: (i, 0)),
    out_shape=jax.ShapeDtypeStruct((ROWS, COLS), jnp.float32),
)


def kernel(dividend, divisor):
    sc_out = _cordiv_sc(dividend, divisor)
    tc_out = _cordiv_tc(dividend, divisor)
    return lax.dynamic_update_slice(tc_out, sc_out, (R_TC, 0))


# speedup vs baseline: 1.0366x; 1.0366x over previous
import functools

import jax
import jax.numpy as jnp
from jax import lax
from jax.experimental import pallas as pl
from jax.experimental.pallas import tpu as pltpu
from jax.experimental.pallas import tpu_sc as plsc

NC, NS, L = 2, 16, 16
NW = NC * NS
ROWS, COLS = 2048, 4096
R_TC = 1920
R_SC = ROWS - R_TC
BR, BC = 8, 2048
TOTAL_BLK = (R_SC // BR) * (COLS // BC)
NBLK = TOTAL_BLK // NW
NSLOT = min(2, NBLK)
TBR = 320

_mesh = plsc.VectorSubcoreMesh(
    core_axis_name="c", subcore_axis_name="s", num_cores=NC, num_subcores=NS
)


@functools.partial(
    pl.kernel,
    mesh=_mesh,
    out_type=jax.ShapeDtypeStruct((R_SC, COLS), jnp.float32),
    scratch_types=[
        pltpu.VMEM((NSLOT, BR, BC), jnp.float32),
        pltpu.VMEM((NSLOT, BR, BC), jnp.int32),
        pltpu.VMEM((NSLOT, BR, BC), jnp.float32),
    ]
    + [pltpu.SemaphoreType.DMA] * (3 * NSLOT),
    compiler_params=pltpu.CompilerParams(use_tc_tiling_on_sc=True),
)
def _cordiv_sc(div_hbm, dsr_hbm, out_hbm, a_v, b_v, o_v, *sems):
    wid = lax.axis_index("s") * NC + lax.axis_index("c")
    sa = sems[0:NSLOT]
    sb = sems[NSLOT : 2 * NSLOT]
    so = sems[2 * NSLOT : 3 * NSLOT]
    zeros = jnp.zeros((L,), jnp.float32)

    def blk(g):
        i = wid + g * NW
        r0 = (i // (COLS // BC)) * BR
        c0 = (i % (COLS // BC)) * BC
        return r0, pl.ds(c0, BC)

    def in_copies(g, s):
        r0, c = blk(g)
        rin = pl.ds(R_TC + r0, BR)
        ca = pltpu.async_copy(div_hbm.at[rin, c], a_v.at[s], sa[s])
        cb = pltpu.async_copy(dsr_hbm.at[rin, c], b_v.at[s], sb[s])
        return ca, cb

    def out_copy(g, s):
        r0, c = blk(g)
        return pltpu.async_copy(o_v.at[s], out_hbm.at[pl.ds(r0, BR), c], so[s])

    pend_in = [None] * NSLOT
    pend_out = [None] * NSLOT
    for g in range(min(NSLOT, NBLK)):
        pend_in[g] = in_copies(g, g)
    for g in range(NBLK):
        s = g % NSLOT
        ca, cb = pend_in[s]
        ca.wait()
        cb.wait()
        if pend_out[s] is not None:
            pend_out[s].wait()
        a_s = a_v.at[s]
        b_s = b_v.at[s]
        o_s = o_v.at[s]

        @plsc.parallel_loop(0, BC, step=L, unroll=2)
        def _(i):
            for r in range(BR):
                d = a_s[r, pl.ds(i, L)]
                q = b_s[r, pl.ds(i, L)]
                o_s[r, pl.ds(i, L)] = jnp.where(q == 1, d, zeros)

        pend_out[s] = out_copy(g, s)
        if g + NSLOT < NBLK:
            pend_in[s] = in_copies(g + NSLOT, s)

    for s in range(NSLOT):
        if pend_out[s] is not None:
            pend_out[s].wait()


def _tc_body(d_ref, q_ref, o_ref):
    o_ref[...] = jnp.where(q_ref[...] == 1, d_ref[...], 0.0)


_cordiv_tc = pl.pallas_call(
    _tc_body,
    grid=(R_TC // TBR,),
    in_specs=[
        pl.BlockSpec((TBR, COLS), lambda i: (i, 0)),
        pl.BlockSpec((TBR, COLS), lambda i: (i, 0)),
    ],
    out_specs=pl.BlockSpec((TBR, COLS), lambda i: (i, 0)),
    out_shape=jax.ShapeDtypeStruct((ROWS, COLS), jnp.float32),
)


def kernel(dividend, divisor):
    sc_out = _cordiv_sc(dividend, divisor)
    tc_out = _cordiv_tc(dividend, divisor)
    return lax.dynamic_update_slice(tc_out, sc_out, (R_TC, 0))
